# baseline (speedup 1.0000x reference)
import functools

import jax
import jax.numpy as jnp
from jax import lax
from jax.experimental import pallas as pl
from jax.experimental.pallas import tpu as pltpu
from jax.experimental.pallas import tpu_sc as plsc

N = 10000
D = 128
E = 320000
NL = 5

NC = 2
NS = 16
NW = NC * NS
BLK = 320
NB = 32
N2 = NB * BLK
DUMP = BLK
ACCR = BLK + 16
SLAB = 4096
EPAY = E + NB * NB * 16 + SLAB + 16

ROWBLK = 1024
GRID = N2 // ROWBLK


def _sc_agg_body(h_hbm, pay_hbm, meta_hbm, out_hbm,
                 acc, hb0, hb1, pay, meta_v, sh0, sh1):
    cid = lax.axis_index("c")
    sid = lax.axis_index("s")
    t = sid * NC + cid

    pltpu.sync_copy(meta_hbm.at[pl.ds(pl.multiple_of(t * 512, 8), 512)], meta_v)

    def _zrow(i, c):
        for k in range(D // 16):
            acc[i, pl.ds(k * 16, 16)] = jnp.zeros((16,), jnp.float32)
        return c
    lax.fori_loop(0, ACCR, _zrow, 0)

    pltpu.async_copy(h_hbm.at[pl.ds(0, BLK)], hb0, sh0)
    pltpu.async_copy(h_hbm.at[pl.ds(BLK, BLK)], hb1, sh1)

    def _seg(s, hb, sem):
        pltpu.make_async_copy(h_hbm.at[pl.ds(0, BLK)], hb, sem).wait()

        mv = meta_v[pl.ds(pl.multiple_of(s * 16, 16), 16)][...]
        base16 = mv[0]
        ln = mv[1]
        nslab = jax.lax.shift_right_logical(ln + (SLAB - 1), 12)

        def _slab(m, c):
            pltpu.sync_copy(
                pay_hbm.at[pl.ds(pl.multiple_of(base16 + m * SLAB, 8), SLAB)],
                pay)
            ngrp = jax.lax.shift_right_logical(
                jnp.minimum(SLAB, ln - m * SLAB), 4)

            def _grp(g, c2):
                pvv = pay[pl.ds(pl.multiple_of(g * 16, 16), 16)][...]
                for l in range(16):
                    pv = pvv[l]
                    sl = jax.lax.shift_right_logical(pv, 9)
                    dl = jnp.bitwise_and(pv, 511)
                    for k in range(D // 16):
                        acc[dl, pl.ds(k * 16, 16)] = (
                            acc[dl, pl.ds(k * 16, 16)]
                            + hb[sl, pl.ds(k * 16, 16)])
                return c2
            lax.fori_loop(0, ngrp, _grp, 0)
            return c
        lax.fori_loop(0, nslab, _slab, 0)

        @pl.when(s + 2 < NB)
        def _():
            pltpu.async_copy(
                h_hbm.at[pl.ds(pl.multiple_of((s + 2) * BLK, 8), BLK)],
                hb, sem)

    def _pair(p, c):
        _seg(2 * p, hb0, sh0)
        _seg(2 * p + 1, hb1, sh1)
        return c
    lax.fori_loop(0, NB // 2, _pair, 0)

    pltpu.sync_copy(acc.at[pl.ds(0, BLK)],
                    out_hbm.at[pl.ds(pl.multiple_of(t * BLK, 8), BLK)])


def _sc_aggregate(h, payload, meta):
    mesh = plsc.VectorSubcoreMesh(
        core_axis_name="c", subcore_axis_name="s", num_cores=NC, num_subcores=NS)
    k = pl.kernel(
        _sc_agg_body,
        out_type=jax.ShapeDtypeStruct((N2, D), jnp.float32),
        mesh=mesh,
        scratch_types=[
            pltpu.VMEM((ACCR, D), jnp.float32),
            pltpu.VMEM((BLK, D), jnp.float32),
            pltpu.VMEM((BLK, D), jnp.float32),
            pltpu.VMEM((SLAB,), jnp.int32),
            pltpu.VMEM((512,), jnp.int32),
            pltpu.SemaphoreType.DMA,
            pltpu.SemaphoreType.DMA,
        ],
    )
    return k(h, payload, meta)


def _enc_body(x_ref, w_ref, b_ref, o_ref):
    z = jnp.dot(x_ref[...], w_ref[...], preferred_element_type=jnp.float32)
    o_ref[...] = jnp.maximum(z + b_ref[...], 0.0)


def _mlp_body(h_ref, p_ref, w1_ref, b1_ref, w2_ref, b2_ref, o_ref):
    z = h_ref[...] + p_ref[...]
    a = jnp.maximum(
        jnp.dot(z, w1_ref[...], preferred_element_type=jnp.float32) + b1_ref[...], 0.0)
    o = jnp.dot(a, w2_ref[...], preferred_element_type=jnp.float32) + b2_ref[...]
    o_ref[...] = jnp.maximum(o, 0.0)


def _mlp_final_body(h_ref, p_ref, w1_ref, b1_ref, w2_ref, b2_ref, o_ref, g_ref):
    z = h_ref[...] + p_ref[...]
    a = jnp.maximum(
        jnp.dot(z, w1_ref[...], preferred_element_type=jnp.float32) + b1_ref[...], 0.0)
    o = jnp.maximum(
        jnp.dot(a, w2_ref[...], preferred_element_type=jnp.float32) + b2_ref[...], 0.0)
    o_ref[...] = o
    i = pl.program_id(0)
    row = lax.broadcasted_iota(jnp.int32, (ROWBLK, D), 0) + i * ROWBLK
    s = jnp.sum(jnp.where(row < N, o, 0.0), axis=0, keepdims=True)

    @pl.when(i == 0)
    def _():
        g_ref[...] = s

    @pl.when(jnp.logical_and(i > 0, i < GRID - 1))
    def _():
        g_ref[...] = g_ref[...] + s

    @pl.when(i == GRID - 1)
    def _():
        g_ref[...] = (g_ref[...] + s) * jnp.float32(1.0 / N)


_ROW_SPEC = pl.BlockSpec((ROWBLK, D), lambda i: (i, 0))
_W_SPEC = pl.BlockSpec((D, D), lambda i: (0, 0))
_B_SPEC = pl.BlockSpec((1, D), lambda i: (0, 0))

_enc_call = pl.pallas_call(
    _enc_body,
    grid=(GRID,),
    in_specs=[_ROW_SPEC, _W_SPEC, _B_SPEC],
    out_specs=_ROW_SPEC,
    out_shape=jax.ShapeDtypeStruct((N2, D), jnp.float32),
)

_mlp_call = pl.pallas_call(
    _mlp_body,
    grid=(GRID,),
    in_specs=[_ROW_SPEC, _ROW_SPEC, _W_SPEC, _B_SPEC, _W_SPEC, _B_SPEC],
    out_specs=_ROW_SPEC,
    out_shape=jax.ShapeDtypeStruct((N2, D), jnp.float32),
)

_mlp_final_call = pl.pallas_call(
    _mlp_final_body,
    grid=(GRID,),
    in_specs=[_ROW_SPEC, _ROW_SPEC, _W_SPEC, _B_SPEC, _W_SPEC, _B_SPEC],
    out_specs=[_ROW_SPEC, pl.BlockSpec((1, D), lambda i: (0, 0))],
    out_shape=[
        jax.ShapeDtypeStruct((N2, D), jnp.float32),
        jax.ShapeDtypeStruct((1, D), jnp.float32),
    ],
)


def kernel(x, edge_index, W_enc, b_enc, W1, b1, W2, b2):
    src = edge_index[0]
    dst = edge_index[1]

    key = (dst // BLK) * jnp.int32(NB) + src // BLK
    order = jnp.argsort(key)
    skey = key[order]
    packed = (src[order] % BLK) * jnp.int32(512) + dst[order] % BLK
    starts = jnp.searchsorted(
        skey, jnp.arange(NB * NB + 1, dtype=jnp.int32)).astype(jnp.int32)
    seg_len = starts[1:] - starts[:-1]
    plen = jnp.bitwise_and(seg_len + 15, jnp.int32(~15))
    pbase = jnp.concatenate(
        [jnp.zeros((1,), jnp.int32), jnp.cumsum(plen).astype(jnp.int32)[:-1]])
    rank = jnp.arange(E, dtype=jnp.int32) - jnp.take(starts[:-1], skey)
    pos = jnp.take(pbase, skey) + rank
    payload = jnp.full((EPAY,), jnp.int32(DUMP), jnp.int32).at[pos].set(packed)
    meta = jnp.zeros((NB, NB, 16), jnp.int32)
    meta = meta.at[:, :, 0].set(pbase.reshape(NB, NB))
    meta = meta.at[:, :, 1].set(plen.reshape(NB, NB))
    meta = meta.reshape(-1)

    xp = jnp.concatenate([x, jnp.zeros((N2 - N, x.shape[1]), x.dtype)])
    h0 = _enc_call(xp, W_enc, b_enc.reshape(1, D))
    h = h0
    gsum = None
    for l in range(NL):
        agg = _sc_aggregate(h, payload, meta)
        b1l = b1[l].reshape(1, D)
        b2l = b2[l].reshape(1, D)
        if l < NL - 1:
            h = _mlp_call(h, agg, W1[l], b1l, W2[l], b2l)
        else:
            h, gsum = _mlp_final_call(h, agg, W1[l], b1l, W2[l], b2l)
    return h[:N], gsum.reshape(D), h0[:N]


# speedup vs baseline: 3.5911x; 3.5911x over previous
import functools

import jax
import jax.numpy as jnp
from jax import lax
from jax.experimental import pallas as pl
from jax.experimental.pallas import tpu as pltpu
from jax.experimental.pallas import tpu_sc as plsc

N = 10000
D = 128
E = 320000
NL = 5

NC = 2
NS = 16
NW = NC * NS
CHUNK = 128
CHUNKS = 80
EPW = CHUNK * CHUNKS
EPAD = NW * EPW
NPAD = 10112
ZROWS = NPAD // NS
OROWS = 624

ROWBLK = 1000
GRID = N // ROWBLK


def _sc_agg_body(h_hbm, pidx_hbm, out_hbm, acc, pidx, rows0, rows1,
                 gsrc0, gdst0, gsrc1, gdst1, sem0, sem1):
    cid = lax.axis_index("c")
    sid = lax.axis_index("s")
    wid = sid * NC + cid
    cbase = wid * CHUNKS

    pltpu.sync_copy(pidx_hbm.at[pl.ds(cbase, CHUNKS)], pidx)

    def _zrow(i, c):
        for k in range(D // 16):
            rows0[i, pl.ds(k * 16, 16)] = jnp.zeros((16,), jnp.float32)
        return c
    lax.fori_loop(0, CHUNK, _zrow, 0)
    zbase = sid * ZROWS
    nfull = ZROWS // CHUNK
    for c in range(nfull):
        pltpu.sync_copy(rows0, acc.at[pl.ds(zbase + c * CHUNK, CHUNK)])
    rem = ZROWS - nfull * CHUNK
    if rem:
        pltpu.sync_copy(rows0.at[pl.ds(0, rem)], acc.at[pl.ds(zbase + nfull * CHUNK, rem)])

    plsc.subcore_barrier()

    def _unpack(j, gsrc, gdst):
        for k in range(CHUNK // 16):
            v = pidx[j, pl.ds(k * 16, 16)]
            gsrc[pl.ds(k * 16, 16)] = jax.lax.shift_right_logical(v, 14)
            gdst[pl.ds(k * 16, 16)] = jnp.bitwise_and(v, 16383)

    _unpack(0, gsrc0, gdst0)
    pltpu.async_copy(h_hbm.at[gsrc0], rows0, sem0)
    _unpack(1, gsrc1, gdst1)
    pltpu.async_copy(h_hbm.at[gsrc1], rows1, sem1)

    def _step2(g, c):
        j0 = 2 * g
        pltpu.make_async_copy(h_hbm.at[gsrc0], rows0, sem0).wait()
        pltpu.sync_copy(rows0, acc.at[gdst0], add=True)

        @pl.when(j0 + 2 < CHUNKS)
        def _():
            _unpack(j0 + 2, gsrc0, gdst0)
            pltpu.async_copy(h_hbm.at[gsrc0], rows0, sem0)

        pltpu.make_async_copy(h_hbm.at[gsrc1], rows1, sem1).wait()
        pltpu.sync_copy(rows1, acc.at[gdst1], add=True)

        @pl.when(j0 + 3 < CHUNKS)
        def _():
            _unpack(j0 + 3, gsrc1, gdst1)
            pltpu.async_copy(h_hbm.at[gsrc1], rows1, sem1)
        return c
    lax.fori_loop(0, CHUNKS // 2, _step2, 0)

    plsc.subcore_barrier()

    obase = sid * OROWS
    pltpu.sync_copy(acc.at[pl.ds(obase, OROWS)], out_hbm.at[cid, pl.ds(obase, OROWS)])

    @pl.when(sid == NS - 1)
    def _():
        tail = NS * OROWS
        pltpu.sync_copy(acc.at[pl.ds(tail, N - tail)],
                        out_hbm.at[cid, pl.ds(tail, N - tail)])


def _sc_aggregate(h, pidx2d):
    mesh = plsc.VectorSubcoreMesh(
        core_axis_name="c", subcore_axis_name="s", num_cores=NC, num_subcores=NS)
    k = pl.kernel(
        _sc_agg_body,
        out_type=jax.ShapeDtypeStruct((NC, N, D), jnp.float32),
        mesh=mesh,
        scratch_types=[
            pltpu.VMEM_SHARED((NPAD, D), jnp.float32),
            pltpu.VMEM((CHUNKS, CHUNK), jnp.int32),
            pltpu.VMEM((CHUNK, D), jnp.float32),
            pltpu.VMEM((CHUNK, D), jnp.float32),
            pltpu.VMEM((CHUNK,), jnp.int32),
            pltpu.VMEM((CHUNK,), jnp.int32),
            pltpu.VMEM((CHUNK,), jnp.int32),
            pltpu.VMEM((CHUNK,), jnp.int32),
            pltpu.SemaphoreType.DMA,
            pltpu.SemaphoreType.DMA,
        ],
    )
    return k(h, pidx2d)


def _enc_body(x_ref, w_ref, b_ref, o_ref):
    z = jnp.dot(x_ref[...], w_ref[...], preferred_element_type=jnp.float32)
    o_ref[...] = jnp.maximum(z + b_ref[...], 0.0)


def _mlp_body(h_ref, p_ref, w1_ref, b1_ref, w2_ref, b2_ref, o_ref):
    z = h_ref[...] + p_ref[0] + p_ref[1]
    a = jnp.maximum(
        jnp.dot(z, w1_ref[...], preferred_element_type=jnp.float32) + b1_ref[...], 0.0)
    o = jnp.dot(a, w2_ref[...], preferred_element_type=jnp.float32) + b2_ref[...]
    o_ref[...] = jnp.maximum(o, 0.0)


def _mlp_final_body(h_ref, p_ref, w1_ref, b1_ref, w2_ref, b2_ref, o_ref, g_ref):
    z = h_ref[...] + p_ref[0] + p_ref[1]
    a = jnp.maximum(
        jnp.dot(z, w1_ref[...], preferred_element_type=jnp.float32) + b1_ref[...], 0.0)
    o = jnp.maximum(
        jnp.dot(a, w2_ref[...], preferred_element_type=jnp.float32) + b2_ref[...], 0.0)
    o_ref[...] = o
    s = jnp.sum(o, axis=0, keepdims=True)
    i = pl.program_id(0)

    @pl.when(i == 0)
    def _():
        g_ref[...] = s

    @pl.when(jnp.logical_and(i > 0, i < GRID - 1))
    def _():
        g_ref[...] = g_ref[...] + s

    @pl.when(i == GRID - 1)
    def _():
        g_ref[...] = (g_ref[...] + s) * jnp.float32(1.0 / N)


_ROW_SPEC = pl.BlockSpec((ROWBLK, D), lambda i: (i, 0))
_P_SPEC = pl.BlockSpec((NC, ROWBLK, D), lambda i: (0, i, 0))
_W_SPEC = pl.BlockSpec((D, D), lambda i: (0, 0))
_B_SPEC = pl.BlockSpec((1, D), lambda i: (0, 0))

_enc_call = pl.pallas_call(
    _enc_body,
    grid=(GRID,),
    in_specs=[_ROW_SPEC, _W_SPEC, _B_SPEC],
    out_specs=_ROW_SPEC,
    out_shape=jax.ShapeDtypeStruct((N, D), jnp.float32),
)

_mlp_call = pl.pallas_call(
    _mlp_body,
    grid=(GRID,),
    in_specs=[_ROW_SPEC, _P_SPEC, _W_SPEC, _B_SPEC, _W_SPEC, _B_SPEC],
    out_specs=_ROW_SPEC,
    out_shape=jax.ShapeDtypeStruct((N, D), jnp.float32),
)

_mlp_final_call = pl.pallas_call(
    _mlp_final_body,
    grid=(GRID,),
    in_specs=[_ROW_SPEC, _P_SPEC, _W_SPEC, _B_SPEC, _W_SPEC, _B_SPEC],
    out_specs=[_ROW_SPEC, pl.BlockSpec((1, D), lambda i: (0, 0))],
    out_shape=[
        jax.ShapeDtypeStruct((N, D), jnp.float32),
        jax.ShapeDtypeStruct((1, D), jnp.float32),
    ],
)


def kernel(x, edge_index, W_enc, b_enc, W1, b1, W2, b2):
    src = edge_index[0]
    dst = edge_index[1]
    pad = EPAD - E
    packed = src * jnp.int32(16384) + dst
    pidx2d = jnp.concatenate(
        [packed, jnp.full((pad,), N, jnp.int32)]).reshape(NW * CHUNKS, CHUNK)

    h0 = _enc_call(x, W_enc, b_enc.reshape(1, D))
    h = h0
    gsum = None
    for l in range(NL):
        parts = _sc_aggregate(h, pidx2d)
        b1l = b1[l].reshape(1, D)
        b2l = b2[l].reshape(1, D)
        if l < NL - 1:
            h = _mlp_call(h, parts, W1[l], b1l, W2[l], b2l)
        else:
            h, gsum = _mlp_final_call(h, parts, W1[l], b1l, W2[l], b2l)
    return h, gsum.reshape(D), h0
